# baseline (speedup 1.0000x reference)
import functools

import jax
import jax.numpy as jnp
from jax import lax
from jax.experimental import pallas as pl
from jax.experimental.pallas import tpu as pltpu
from jax.experimental.pallas import tpu_sc as plsc

NNZ = 2097152
NBINS = 2048
NC = 2
NS = 16
NW = NC * NS
CHUNK = NNZ // NW
SUB = 8192
NSUB = CHUNK // SUB
VPS = SUB // 16
NVB = NBINS // 16
MASK21 = (1 << 21) - 1

_MESH = plsc.VectorSubcoreMesh(core_axis_name="c", subcore_axis_name="s")
_CP = pltpu.CompilerParams(needs_layout_passes=False)


def _wid():
    return lax.axis_index("s") * NC + lax.axis_index("c")


def _vsl(j):
    return pl.ds(pl.multiple_of(j * 16, 16), 16)


def _zero(ref, nv):
    def body(j, _):
        ref[_vsl(j)] = jnp.zeros((16,), jnp.int32)
        return 0
    lax.fori_loop(0, nv, body, 0)


def _chunk(arr, base, n):
    return arr.at[pl.ds(pl.multiple_of(base, 8), n)]


def _hist_common(src_hbm, out_hbm, fbuf, hist_v, dig):
    w = _wid()
    _zero(hist_v, NVB)
    ones = jnp.ones((16,), jnp.int32)

    def sub(s, _):
        pltpu.sync_copy(_chunk(src_hbm, w * CHUNK + s * SUB, SUB), fbuf)

        def body(j, _):
            d = dig(fbuf[_vsl(j)])
            plsc.addupdate_scatter(hist_v, [d], ones)
            return 0
        lax.fori_loop(0, VPS, body, 0)
        return 0
    lax.fori_loop(0, NSUB, sub, 0)
    pltpu.sync_copy(hist_v, out_hbm.at[w])


def _dig_id(x):
    return x & (NBINS - 1)


def _dig_hi(x):
    return (x >> 21) & (NBINS - 1)


def _offsets(hist_hbm, row_v, tot_v, off_v, spl_ref=None):
    w = _wid()
    _zero(tot_v, NVB)
    _zero(off_v, NVB)

    def trow(t, _):
        pltpu.sync_copy(hist_hbm.at[t], row_v)
        m = (t < w).astype(jnp.int32)

        def inner(j, _):
            sl = _vsl(j)
            row = row_v[sl]
            tot_v[sl] = tot_v[sl] + row
            off_v[sl] = off_v[sl] + row * m
            return 0
        lax.fori_loop(0, NVB, inner, 0)
        return 0
    lax.fori_loop(0, NW, trow, 0)

    fifteen = jnp.full((16,), 15, jnp.int32)

    def scan(j, carry):
        sl = _vsl(j)
        v = tot_v[sl]
        cs = plsc.cumsum(v)
        excl = cs - v + carry
        if spl_ref is not None:
            spl_ref[sl] = excl
        off_v[sl] = off_v[sl] + excl
        return carry + jnp.take(cs, fifteen)
    lax.fori_loop(0, NVB, scan, jnp.zeros((16,), jnp.int32))


def _rank(off_v, d):
    cnt, lastm = plsc.scan_count(d)
    cnt = cnt.astype(jnp.int32)
    cur = plsc.load_gather(off_v, [d])
    plsc.store_scatter(off_v, [d], cur + cnt, mask=lastm)
    return cur + cnt - 1


@functools.partial(
    pl.kernel, mesh=_MESH, compiler_params=_CP,
    out_type=jax.ShapeDtypeStruct((NW, NBINS), jnp.int32),
    scratch_types=[pltpu.VMEM((SUB,), jnp.int32),
                   pltpu.VMEM((NBINS,), jnp.int32)],
)
def _hist_a(B, out, fbuf, hist_v):
    _hist_common(B, out, fbuf, hist_v, _dig_id)


@functools.partial(
    pl.kernel, mesh=_MESH, compiler_params=_CP,
    out_type=jax.ShapeDtypeStruct((NW, NBINS), jnp.int32),
    scratch_types=[pltpu.VMEM((SUB,), jnp.int32),
                   pltpu.VMEM((NBINS,), jnp.int32)],
)
def _hist_hi(X, out, fbuf, hist_v):
    _hist_common(X, out, fbuf, hist_v, _dig_hi)


@functools.partial(
    pl.kernel, mesh=_MESH, compiler_params=_CP,
    out_type=(jax.ShapeDtypeStruct((NNZ,), jnp.int32),
              jax.ShapeDtypeStruct((NBINS,), jnp.int32)),
    scratch_types=([pltpu.VMEM((SUB,), jnp.int32),
                    pltpu.VMEM((SUB,), jnp.int32)]
                   + [pltpu.VMEM((SUB,), jnp.int32)] * 8
                   + [pltpu.VMEM((NBINS,), jnp.int32)] * 4
                   + [pltpu.SemaphoreType.DMA] * 4),
)
def _perm_a(B, R, hA, W1, SPL, bbuf, rbuf,
            wb0, wb1, wb2, wb3, db0, db1, db2, db3,
            row_v, tot_v, off_v, spl_v, s0, s1, s2, s3):
    w = _wid()
    _offsets(hA, row_v, tot_v, off_v, spl_v)
    iota = lax.iota(jnp.int32, 16)
    wbufs, dbufs, sems = [wb0, wb1, wb2, wb3], [db0, db1, db2, db3], \
        [s0, s1, s2, s3]

    @pl.when(w == 0)
    def _():
        pltpu.sync_copy(spl_v, SPL)

    pend = [None] * 4
    for s in range(NSUB):
        slot = s % 4
        base = w * CHUNK + s * SUB
        pltpu.sync_copy(_chunk(B, base, SUB), bbuf)
        pltpu.sync_copy(_chunk(R, base, SUB), rbuf)
        if pend[slot] is not None:
            pend[slot].wait()
        wbuf, dbuf = wbufs[slot], dbufs[slot]

        def body(j, _, base=base, wbuf=wbuf, dbuf=dbuf):
            sl = _vsl(j)
            b = bbuf[sl]
            k2 = rbuf[sl] + plsc.load_gather(spl_v, [b])
            dbuf[sl] = _rank(off_v, b)
            wbuf[sl] = ((k2 & (NBINS - 1)) << 21) | (base + j * 16 + iota)
            return 0
        lax.fori_loop(0, VPS, body, 0)
        pend[slot] = pltpu.async_copy(wbuf, W1.at[dbuf], sems[slot])
    for p in pend:
        if p is not None:
            p.wait()


@functools.partial(
    pl.kernel, mesh=_MESH, compiler_params=_CP,
    out_type=jax.ShapeDtypeStruct((NNZ,), jnp.int32),
    scratch_types=([pltpu.VMEM((SUB,), jnp.int32)] * 4
                   + [pltpu.VMEM((SUB,), jnp.int32)] * 8
                   + [pltpu.VMEM((NBINS,), jnp.int32)] * 4
                   + [pltpu.SemaphoreType.DMA]
                   + [pltpu.SemaphoreType.DMA] * 4),
)
def _perm_b(W1, B, R, SPL, hB, P2, wbuf, ibuf, bgbuf, rgbuf,
            pb0, pb1, pb2, pb3, db0, db1, db2, db3,
            row_v, tot_v, off_v, spl_v, gsem, s0, s1, s2, s3):
    w = _wid()
    _offsets(hB, row_v, tot_v, off_v)
    pltpu.sync_copy(SPL, spl_v)
    pbufs, dbufs, sems = [pb0, pb1, pb2, pb3], [db0, db1, db2, db3], \
        [s0, s1, s2, s3]

    pend = [None] * 4
    for s in range(NSUB):
        slot = s % 4
        base = w * CHUNK + s * SUB
        pltpu.sync_copy(_chunk(W1, base, SUB), wbuf)

        def ext(j, _):
            sl = _vsl(j)
            ibuf[sl] = wbuf[sl] & MASK21
            return 0
        lax.fori_loop(0, VPS, ext, 0)
        c0 = pltpu.async_copy(B.at[ibuf], bgbuf, gsem)
        c1 = pltpu.async_copy(R.at[ibuf], rgbuf, gsem)
        c0.wait()
        c1.wait()
        if pend[slot] is not None:
            pend[slot].wait()
        pbuf, dbuf = pbufs[slot], dbufs[slot]

        def body(j, _, pbuf=pbuf, dbuf=dbuf):
            sl = _vsl(j)
            d = (wbuf[sl] >> 21) & (NBINS - 1)
            k2 = rgbuf[sl] + plsc.load_gather(spl_v, [bgbuf[sl]])
            dbuf[sl] = _rank(off_v, d)
            pbuf[sl] = ((k2 >> 11) << 21) | ibuf[sl]
            return 0
        lax.fori_loop(0, VPS, body, 0)
        pend[slot] = pltpu.async_copy(pbuf, P2.at[dbuf], sems[slot])
    for p in pend:
        if p is not None:
            p.wait()


@functools.partial(
    pl.kernel, mesh=_MESH, compiler_params=_CP,
    out_type=jax.ShapeDtypeStruct((NNZ,), jnp.int32),
    scratch_types=([pltpu.VMEM((SUB,), jnp.int32)]
                   + [pltpu.VMEM((SUB,), jnp.int32)] * 8
                   + [pltpu.VMEM((NBINS,), jnp.int32)] * 3
                   + [pltpu.SemaphoreType.DMA] * 4),
)
def _perm_c(P2, hC, I3, pfbuf,
            ob0, ob1, ob2, ob3, db0, db1, db2, db3,
            row_v, tot_v, off_v, s0, s1, s2, s3):
    w = _wid()
    _offsets(hC, row_v, tot_v, off_v)
    obufs, dbufs, sems = [ob0, ob1, ob2, ob3], [db0, db1, db2, db3], \
        [s0, s1, s2, s3]

    pend = [None] * 4
    for s in range(NSUB):
        slot = s % 4
        base = w * CHUNK + s * SUB
        pltpu.sync_copy(_chunk(P2, base, SUB), pfbuf)
        if pend[slot] is not None:
            pend[slot].wait()
        obuf, dbuf = obufs[slot], dbufs[slot]

        def body(j, _, obuf=obuf, dbuf=dbuf):
            sl = _vsl(j)
            x = pfbuf[sl]
            d = (x >> 21) & (NBINS - 1)
            dbuf[sl] = _rank(off_v, d)
            obuf[sl] = x & MASK21
            return 0
        lax.fori_loop(0, VPS, body, 0)
        pend[slot] = pltpu.async_copy(obuf, I3.at[dbuf], sems[slot])
    for p in pend:
        if p is not None:
            p.wait()


@functools.partial(
    pl.kernel, mesh=_MESH, compiler_params=_CP,
    out_type=(jax.ShapeDtypeStruct((2 * NNZ,), jnp.int32),
              jax.ShapeDtypeStruct((NNZ,), jnp.int32),
              jax.ShapeDtypeStruct((NNZ,), jnp.float32)),
    scratch_types=[pltpu.VMEM((SUB,), jnp.int32),
                   pltpu.VMEM((SUB,), jnp.int32),
                   pltpu.VMEM((SUB,), jnp.int32),
                   pltpu.VMEM((SUB,), jnp.int32),
                   pltpu.VMEM((SUB,), jnp.float32),
                   pltpu.VMEM((2 * SUB,), jnp.int32),
                   pltpu.SemaphoreType.DMA],
)
def _final(I3, B, R, C, V, EI2, RID, EW, ibuf, bb, rb, cb, vb, eibuf, sem):
    w = _wid()
    iota = lax.iota(jnp.int32, 16)

    def sub(s, _):
        base = w * CHUNK + s * SUB
        pltpu.sync_copy(_chunk(I3, base, SUB), ibuf)
        c0 = pltpu.async_copy(B.at[ibuf], bb, sem)
        c1 = pltpu.async_copy(R.at[ibuf], rb, sem)
        c2 = pltpu.async_copy(C.at[ibuf], cb, sem)
        c3 = pltpu.async_copy(V.at[ibuf], vb, sem)
        c0.wait()
        c1.wait()
        c2.wait()
        c3.wait()

        def ilv(j, _):
            sl = _vsl(j)
            lidx = j * 16 + iota
            plsc.store_scatter(eibuf, [lidx * 2], rb[sl])
            plsc.store_scatter(eibuf, [lidx * 2 + 1], cb[sl])
            return 0
        lax.fori_loop(0, VPS, ilv, 0)
        pltpu.sync_copy(eibuf, _chunk(EI2, base * 2, SUB * 2))
        pltpu.sync_copy(bb, _chunk(RID, base, SUB))
        pltpu.sync_copy(vb, _chunk(EW, base, SUB))
        return 0
    lax.fori_loop(0, NSUB, sub, 0)


def kernel(indices, values):
    B = indices[:, 0]
    R = indices[:, 1]
    C = indices[:, 2]
    hA = _hist_a(B)
    W1, SPL = _perm_a(B, R, hA)
    hB = _hist_hi(W1)
    P2 = _perm_b(W1, B, R, SPL, hB)
    hC = _hist_hi(P2)
    I3 = _perm_c(P2, hC)
    EI2, RID, EW = _final(I3, B, R, C, values)
    return EI2.reshape(NNZ, 2), RID, EW[:, None]


# speedup vs baseline: 1.2956x; 1.2956x over previous
import functools

import jax
import jax.numpy as jnp
from jax import lax
from jax.experimental import pallas as pl
from jax.experimental.pallas import tpu as pltpu
from jax.experimental.pallas import tpu_sc as plsc

NNZ = 2097152
NBINS = 2048
NC = 2
NS = 16
NW = NC * NS
CHUNK = NNZ // NW
SUB = 8192
NSUB = CHUNK // SUB
VPS = SUB // 16
NVB = NBINS // 16
MASK21 = (1 << 21) - 1

_MESH = plsc.VectorSubcoreMesh(core_axis_name="c", subcore_axis_name="s")
_CP = pltpu.CompilerParams(needs_layout_passes=False)


def _wid():
    return lax.axis_index("s") * NC + lax.axis_index("c")


def _vsl(j):
    return pl.ds(pl.multiple_of(j * 16, 16), 16)


def _zero(ref, nv):
    def body(j, _):
        ref[_vsl(j)] = jnp.zeros((16,), jnp.int32)
        return 0
    lax.fori_loop(0, nv, body, 0)


def _chunk(arr, base, n):
    return arr.at[pl.ds(pl.multiple_of(base, 8), n)]


def _hist_common(src_hbm, out_hbm, fbuf, hist_v, dig):
    w = _wid()
    _zero(hist_v, NVB)
    ones = jnp.ones((16,), jnp.int32)

    def sub(s, _):
        pltpu.sync_copy(_chunk(src_hbm, w * CHUNK + s * SUB, SUB), fbuf)

        def body(j, _):
            d = dig(fbuf[_vsl(j)])
            plsc.addupdate_scatter(hist_v, [d], ones)
            return 0
        lax.fori_loop(0, VPS, body, 0)
        return 0
    lax.fori_loop(0, NSUB, sub, 0)
    pltpu.sync_copy(hist_v, out_hbm.at[w])


def _dig_id(x):
    return x & (NBINS - 1)


def _dig_hi(x):
    return (x >> 21) & (NBINS - 1)


def _offsets(hist_hbm, row_v, tot_v, off_v, spl_ref=None, upto=None):
    w = _wid() if upto is None else upto
    _zero(tot_v, NVB)
    _zero(off_v, NVB)

    def trow(t, _):
        pltpu.sync_copy(hist_hbm.at[t], row_v)
        m = (t < w).astype(jnp.int32)

        def inner(j, _):
            sl = _vsl(j)
            row = row_v[sl]
            tot_v[sl] = tot_v[sl] + row
            off_v[sl] = off_v[sl] + row * m
            return 0
        lax.fori_loop(0, NVB, inner, 0)
        return 0
    lax.fori_loop(0, NW, trow, 0)

    fifteen = jnp.full((16,), 15, jnp.int32)

    def scan(j, carry):
        sl = _vsl(j)
        v = tot_v[sl]
        cs = plsc.cumsum(v)
        excl = cs - v + carry
        if spl_ref is not None:
            spl_ref[sl] = excl
        off_v[sl] = off_v[sl] + excl
        return carry + jnp.take(cs, fifteen)
    lax.fori_loop(0, NVB, scan, jnp.zeros((16,), jnp.int32))


def _rank(off_v, d):
    cnt, lastm = plsc.scan_count(d)
    cnt = cnt.astype(jnp.int32)
    cur = plsc.load_gather(off_v, [d])
    plsc.store_scatter(off_v, [d], cur + cnt, mask=lastm)
    return cur + cnt - 1


@functools.partial(
    pl.kernel, mesh=_MESH, compiler_params=_CP,
    out_type=jax.ShapeDtypeStruct((NW, NBINS), jnp.int32),
    scratch_types=[pltpu.VMEM((SUB,), jnp.int32),
                   pltpu.VMEM((NBINS,), jnp.int32)],
)
def _hist_a(B, out, fbuf, hist_v):
    _hist_common(B, out, fbuf, hist_v, _dig_id)


@functools.partial(
    pl.kernel, mesh=_MESH, compiler_params=_CP,
    out_type=jax.ShapeDtypeStruct((NW, NBINS), jnp.int32),
    scratch_types=[pltpu.VMEM((SUB,), jnp.int32),
                   pltpu.VMEM((NBINS,), jnp.int32)],
)
def _hist_hi(X, out, fbuf, hist_v):
    _hist_common(X, out, fbuf, hist_v, _dig_hi)


@functools.partial(
    pl.kernel, mesh=_MESH, compiler_params=_CP,
    out_type=(jax.ShapeDtypeStruct((NNZ,), jnp.int32),
              jax.ShapeDtypeStruct((NBINS,), jnp.int32)),
    scratch_types=([pltpu.VMEM((SUB,), jnp.int32),
                    pltpu.VMEM((SUB,), jnp.int32)]
                   + [pltpu.VMEM((SUB,), jnp.int32)] * 8
                   + [pltpu.VMEM((NBINS,), jnp.int32)] * 4
                   + [pltpu.SemaphoreType.DMA] * 4),
)
def _perm_a(B, R, hA, W1, SPL, bbuf, rbuf,
            wb0, wb1, wb2, wb3, db0, db1, db2, db3,
            row_v, tot_v, off_v, spl_v, s0, s1, s2, s3):
    w = _wid()
    _offsets(hA, row_v, tot_v, off_v, spl_v)
    iota = lax.iota(jnp.int32, 16)
    wbufs, dbufs, sems = [wb0, wb1, wb2, wb3], [db0, db1, db2, db3], \
        [s0, s1, s2, s3]

    @pl.when(w == 0)
    def _():
        pltpu.sync_copy(spl_v, SPL)

    pend = [None] * 4
    for s in range(NSUB):
        slot = s % 4
        base = w * CHUNK + s * SUB
        pltpu.sync_copy(_chunk(B, base, SUB), bbuf)
        pltpu.sync_copy(_chunk(R, base, SUB), rbuf)
        if pend[slot] is not None:
            pend[slot].wait()
        wbuf, dbuf = wbufs[slot], dbufs[slot]

        def body(j, _, base=base, wbuf=wbuf, dbuf=dbuf):
            sl = _vsl(j)
            b = bbuf[sl]
            k2 = rbuf[sl] + plsc.load_gather(spl_v, [b])
            dbuf[sl] = _rank(off_v, b)
            wbuf[sl] = ((k2 & (NBINS - 1)) << 21) | (base + j * 16 + iota)
            return 0
        lax.fori_loop(0, VPS, body, 0)
        pend[slot] = pltpu.async_copy(wbuf, W1.at[dbuf], sems[slot])
    for p in pend:
        if p is not None:
            p.wait()


@functools.partial(
    pl.kernel, mesh=_MESH, compiler_params=_CP,
    out_type=jax.ShapeDtypeStruct((NNZ,), jnp.int32),
    scratch_types=([pltpu.VMEM((SUB,), jnp.int32)] * 4
                   + [pltpu.VMEM((SUB,), jnp.int32)] * 8
                   + [pltpu.VMEM((NBINS,), jnp.int32)] * 4
                   + [pltpu.SemaphoreType.DMA]
                   + [pltpu.SemaphoreType.DMA] * 4),
)
def _perm_b(W1, B, R, SPL, hB, P2, wbuf, ibuf, bgbuf, rgbuf,
            pb0, pb1, pb2, pb3, db0, db1, db2, db3,
            row_v, tot_v, off_v, spl_v, gsem, s0, s1, s2, s3):
    w = _wid()
    _offsets(hB, row_v, tot_v, off_v)
    pltpu.sync_copy(SPL, spl_v)
    pbufs, dbufs, sems = [pb0, pb1, pb2, pb3], [db0, db1, db2, db3], \
        [s0, s1, s2, s3]

    pend = [None] * 4
    for s in range(NSUB):
        slot = s % 4
        base = w * CHUNK + s * SUB
        pltpu.sync_copy(_chunk(W1, base, SUB), wbuf)

        def ext(j, _):
            sl = _vsl(j)
            ibuf[sl] = wbuf[sl] & MASK21
            return 0
        lax.fori_loop(0, VPS, ext, 0)
        c0 = pltpu.async_copy(B.at[ibuf], bgbuf, gsem)
        c1 = pltpu.async_copy(R.at[ibuf], rgbuf, gsem)
        c0.wait()
        c1.wait()
        if pend[slot] is not None:
            pend[slot].wait()
        pbuf, dbuf = pbufs[slot], dbufs[slot]

        def body(j, _, pbuf=pbuf, dbuf=dbuf):
            sl = _vsl(j)
            d = (wbuf[sl] >> 21) & (NBINS - 1)
            k2 = rgbuf[sl] + plsc.load_gather(spl_v, [bgbuf[sl]])
            dbuf[sl] = _rank(off_v, d)
            pbuf[sl] = ((k2 >> 11) << 21) | ibuf[sl]
            return 0
        lax.fori_loop(0, VPS, body, 0)
        pend[slot] = pltpu.async_copy(pbuf, P2.at[dbuf], sems[slot])
    for p in pend:
        if p is not None:
            p.wait()


@functools.partial(
    pl.kernel, mesh=_MESH, compiler_params=_CP,
    out_type=jax.ShapeDtypeStruct((NNZ,), jnp.int32),
    scratch_types=([pltpu.VMEM((SUB,), jnp.int32)] * 3
                   + [pltpu.VMEM_SHARED((NNZ // NC + 16,), jnp.int32)]
                   + [pltpu.VMEM((NBINS,), jnp.int32)] * 3
                   + [pltpu.SemaphoreType.DMA]),
)
def _perm_c(P2, hC, I3, pfbuf, obuf, dbuf, sp, row_v, tot_v, off_v, sem):
    cid = lax.axis_index("c")
    tid = lax.axis_index("s")
    CH2 = NNZ // NS
    HALF = NNZ // NC
    _offsets(hC, row_v, tot_v, off_v, upto=2 * tid)
    lo = cid * HALF
    iota = lax.iota(jnp.int32, 16)

    def sub(s, _):
        base = tid * CH2 + s * SUB
        pltpu.sync_copy(_chunk(P2, base, SUB), pfbuf)

        def body(j, _):
            sl = _vsl(j)
            x = pfbuf[sl]
            d = (x >> 21) & (NBINS - 1)
            dest = _rank(off_v, d) - lo
            inr = (dest >= 0) & (dest < HALF)
            dbuf[sl] = jnp.where(inr, dest, HALF + iota)
            obuf[sl] = x & MASK21
            return 0
        lax.fori_loop(0, VPS, body, 0)
        pltpu.async_copy(obuf, sp.at[dbuf], sem).wait()
        return 0
    lax.fori_loop(0, CH2 // SUB, sub, 0)
    plsc.subcore_barrier()
    SL = HALF // NS
    pltpu.sync_copy(sp.at[pl.ds(tid * SL, SL)],
                    I3.at[pl.ds(pl.multiple_of(lo + tid * SL, 8), SL)])


@functools.partial(
    pl.kernel, mesh=_MESH, compiler_params=_CP,
    out_type=(jax.ShapeDtypeStruct((2 * NNZ,), jnp.int32),
              jax.ShapeDtypeStruct((NNZ,), jnp.int32),
              jax.ShapeDtypeStruct((NNZ,), jnp.float32)),
    scratch_types=[pltpu.VMEM((SUB,), jnp.int32),
                   pltpu.VMEM((SUB,), jnp.int32),
                   pltpu.VMEM((SUB,), jnp.int32),
                   pltpu.VMEM((SUB,), jnp.int32),
                   pltpu.VMEM((SUB,), jnp.float32),
                   pltpu.VMEM((2 * SUB,), jnp.int32),
                   pltpu.SemaphoreType.DMA],
)
def _final(I3, B, R, C, V, EI2, RID, EW, ibuf, bb, rb, cb, vb, eibuf, sem):
    w = _wid()
    iota = lax.iota(jnp.int32, 16)

    def sub(s, _):
        base = w * CHUNK + s * SUB
        pltpu.sync_copy(_chunk(I3, base, SUB), ibuf)
        c0 = pltpu.async_copy(B.at[ibuf], bb, sem)
        c1 = pltpu.async_copy(R.at[ibuf], rb, sem)
        c2 = pltpu.async_copy(C.at[ibuf], cb, sem)
        c3 = pltpu.async_copy(V.at[ibuf], vb, sem)
        c0.wait()
        c1.wait()
        c2.wait()
        c3.wait()

        def ilv(j, _):
            sl = _vsl(j)
            lidx = j * 16 + iota
            plsc.store_scatter(eibuf, [lidx * 2], rb[sl])
            plsc.store_scatter(eibuf, [lidx * 2 + 1], cb[sl])
            return 0
        lax.fori_loop(0, VPS, ilv, 0)
        pltpu.sync_copy(eibuf, _chunk(EI2, base * 2, SUB * 2))
        pltpu.sync_copy(bb, _chunk(RID, base, SUB))
        pltpu.sync_copy(vb, _chunk(EW, base, SUB))
        return 0
    lax.fori_loop(0, NSUB, sub, 0)


def kernel(indices, values):
    B = indices[:, 0]
    R = indices[:, 1]
    C = indices[:, 2]
    hA = _hist_a(B)
    W1, SPL = _perm_a(B, R, hA)
    hB = _hist_hi(W1)
    P2 = _perm_b(W1, B, R, SPL, hB)
    hC = _hist_hi(P2)
    I3 = _perm_c(P2, hC)
    EI2, RID, EW = _final(I3, B, R, C, values)
    return EI2.reshape(NNZ, 2), RID, EW[:, None]


# speedup vs baseline: 2.7864x; 2.1507x over previous
import functools

import jax
import jax.numpy as jnp
from jax import lax
from jax.experimental import pallas as pl
from jax.experimental.pallas import tpu as pltpu
from jax.experimental.pallas import tpu_sc as plsc

NNZ = 2097152
NBINS = 2048
NC = 2
NS = 16
NW = NC * NS
CHUNK = NNZ // NW
SUB = 8192
NSUB = CHUNK // SUB
VPS = SUB // 16
NVB = NBINS // 16
MASK21 = (1 << 21) - 1

_MESH = plsc.VectorSubcoreMesh(core_axis_name="c", subcore_axis_name="s")
_CP = pltpu.CompilerParams(needs_layout_passes=False)


def _wid():
    return lax.axis_index("s") * NC + lax.axis_index("c")


def _vsl(j):
    return pl.ds(pl.multiple_of(j * 16, 16), 16)


def _zero(ref, nv):
    def body(j, _):
        ref[_vsl(j)] = jnp.zeros((16,), jnp.int32)
        return 0
    lax.fori_loop(0, nv, body, 0)


def _chunk(arr, base, n):
    return arr.at[pl.ds(pl.multiple_of(base, 8), n)]


def _hist_common(src_hbm, out_hbm, fbuf, hist_v, dig):
    w = _wid()
    _zero(hist_v, NVB)
    ones = jnp.ones((16,), jnp.int32)

    def sub(s, _):
        pltpu.sync_copy(_chunk(src_hbm, w * CHUNK + s * SUB, SUB), fbuf)

        def body(j, _):
            d = dig(fbuf[_vsl(j)])
            plsc.addupdate_scatter(hist_v, [d], ones)
            return 0
        lax.fori_loop(0, VPS, body, 0)
        return 0
    lax.fori_loop(0, NSUB, sub, 0)
    pltpu.sync_copy(hist_v, out_hbm.at[w])


def _dig_id(x):
    return x & (NBINS - 1)


def _dig_hi(x):
    return (x >> 21) & (NBINS - 1)


def _offsets(hist_hbm, row_v, tot_v, off_v, spl_ref=None, upto=None):
    w = _wid() if upto is None else upto
    _zero(tot_v, NVB)
    _zero(off_v, NVB)

    def trow(t, _):
        pltpu.sync_copy(hist_hbm.at[t], row_v)
        m = (t < w).astype(jnp.int32)

        def inner(j, _):
            sl = _vsl(j)
            row = row_v[sl]
            tot_v[sl] = tot_v[sl] + row
            off_v[sl] = off_v[sl] + row * m
            return 0
        lax.fori_loop(0, NVB, inner, 0)
        return 0
    lax.fori_loop(0, NW, trow, 0)

    fifteen = jnp.full((16,), 15, jnp.int32)

    def scan(j, carry):
        sl = _vsl(j)
        v = tot_v[sl]
        cs = plsc.cumsum(v)
        excl = cs - v + carry
        if spl_ref is not None:
            spl_ref[sl] = excl
        off_v[sl] = off_v[sl] + excl
        return carry + jnp.take(cs, fifteen)
    lax.fori_loop(0, NVB, scan, jnp.zeros((16,), jnp.int32))


def _rank(off_v, d):
    cnt, lastm = plsc.scan_count(d)
    cnt = cnt.astype(jnp.int32)
    cur = plsc.load_gather(off_v, [d])
    plsc.store_scatter(off_v, [d], cur + cnt, mask=lastm)
    return cur + cnt - 1


@functools.partial(
    pl.kernel, mesh=_MESH, compiler_params=_CP,
    out_type=jax.ShapeDtypeStruct((NW, NBINS), jnp.int32),
    scratch_types=[pltpu.VMEM((SUB,), jnp.int32),
                   pltpu.VMEM((NBINS,), jnp.int32)],
)
def _hist_a(B, out, fbuf, hist_v):
    _hist_common(B, out, fbuf, hist_v, _dig_id)


@functools.partial(
    pl.kernel, mesh=_MESH, compiler_params=_CP,
    out_type=jax.ShapeDtypeStruct((NW, NBINS), jnp.int32),
    scratch_types=[pltpu.VMEM((SUB,), jnp.int32),
                   pltpu.VMEM((NBINS,), jnp.int32)],
)
def _hist_hi(X, out, fbuf, hist_v):
    _hist_common(X, out, fbuf, hist_v, _dig_hi)


@functools.partial(
    pl.kernel, mesh=_MESH, compiler_params=_CP,
    out_type=(jax.ShapeDtypeStruct((NNZ,), jnp.int32),
              jax.ShapeDtypeStruct((NBINS,), jnp.int32),
              jax.ShapeDtypeStruct((NNZ,), jnp.int32)),
    scratch_types=([pltpu.VMEM((SUB,), jnp.int32)] * 5
                   + [pltpu.VMEM_SHARED((NNZ // NC + 16,), jnp.int32)]
                   + [pltpu.VMEM((NBINS,), jnp.int32)] * 4
                   + [pltpu.SemaphoreType.DMA]),
)
def _perm_a(B, R, hA, W1, SPL, K2H, bbuf, rbuf, wbuf, dbuf, hbuf, sp,
            row_v, tot_v, off_v, spl_v, sem):
    cid = lax.axis_index("c")
    tid = lax.axis_index("s")
    CH2 = NNZ // NS
    HALF = NNZ // NC
    _offsets(hA, row_v, tot_v, off_v, spl_v, upto=2 * tid)
    lo = cid * HALF
    iota = lax.iota(jnp.int32, 16)

    @pl.when(_wid() == 0)
    def _():
        pltpu.sync_copy(spl_v, SPL)

    def sub(s, _):
        base = tid * CH2 + s * SUB
        pltpu.sync_copy(_chunk(B, base, SUB), bbuf)
        pltpu.sync_copy(_chunk(R, base, SUB), rbuf)

        def body(j, _):
            sl = _vsl(j)
            b = bbuf[sl]
            k2 = rbuf[sl] + plsc.load_gather(spl_v, [b])
            hbuf[sl] = k2 >> 11
            dest = _rank(off_v, b) - lo
            inr = (dest >= 0) & (dest < HALF)
            dbuf[sl] = jnp.where(inr, dest, HALF + iota)
            wbuf[sl] = ((k2 & (NBINS - 1)) << 21) | (base + j * 16 + iota)
            return 0
        lax.fori_loop(0, VPS, body, 0)

        @pl.when(cid == 0)
        def _():
            pltpu.sync_copy(hbuf, _chunk(K2H, base, SUB))
        pltpu.async_copy(wbuf, sp.at[dbuf], sem).wait()
        return 0
    lax.fori_loop(0, CH2 // SUB, sub, 0)
    plsc.subcore_barrier()
    SL = HALF // NS
    pltpu.sync_copy(sp.at[pl.ds(tid * SL, SL)],
                    W1.at[pl.ds(pl.multiple_of(lo + tid * SL, 8), SL)])


@functools.partial(
    pl.kernel, mesh=_MESH, compiler_params=_CP,
    out_type=jax.ShapeDtypeStruct((NNZ,), jnp.int32),
    scratch_types=([pltpu.VMEM((SUB,), jnp.int32)] * 5
                   + [pltpu.VMEM_SHARED((NNZ // NC + 16,), jnp.int32)]
                   + [pltpu.VMEM((NBINS,), jnp.int32)] * 3
                   + [pltpu.SemaphoreType.DMA, pltpu.SemaphoreType.DMA]),
)
def _perm_b(W1, K2H, hB, P2, wbuf, ibuf, hgbuf, pbuf, dbuf, sp,
            row_v, tot_v, off_v, gsem, sem):
    cid = lax.axis_index("c")
    tid = lax.axis_index("s")
    CH2 = NNZ // NS
    HALF = NNZ // NC
    _offsets(hB, row_v, tot_v, off_v, upto=2 * tid)
    lo = cid * HALF
    iota = lax.iota(jnp.int32, 16)

    def sub(s, _):
        base = tid * CH2 + s * SUB
        pltpu.sync_copy(_chunk(W1, base, SUB), wbuf)

        def ext(j, _):
            sl = _vsl(j)
            ibuf[sl] = wbuf[sl] & MASK21
            return 0
        lax.fori_loop(0, VPS, ext, 0)
        pltpu.async_copy(K2H.at[ibuf], hgbuf, gsem).wait()

        def body(j, _):
            sl = _vsl(j)
            d = (wbuf[sl] >> 21) & (NBINS - 1)
            dest = _rank(off_v, d) - lo
            inr = (dest >= 0) & (dest < HALF)
            dbuf[sl] = jnp.where(inr, dest, HALF + iota)
            pbuf[sl] = (hgbuf[sl] << 21) | ibuf[sl]
            return 0
        lax.fori_loop(0, VPS, body, 0)
        pltpu.async_copy(pbuf, sp.at[dbuf], sem).wait()
        return 0
    lax.fori_loop(0, CH2 // SUB, sub, 0)
    plsc.subcore_barrier()
    SL = HALF // NS
    pltpu.sync_copy(sp.at[pl.ds(tid * SL, SL)],
                    P2.at[pl.ds(pl.multiple_of(lo + tid * SL, 8), SL)])


@functools.partial(
    pl.kernel, mesh=_MESH, compiler_params=_CP,
    out_type=jax.ShapeDtypeStruct((NNZ,), jnp.int32),
    scratch_types=([pltpu.VMEM((SUB,), jnp.int32)] * 3
                   + [pltpu.VMEM_SHARED((NNZ // NC + 16,), jnp.int32)]
                   + [pltpu.VMEM((NBINS,), jnp.int32)] * 3
                   + [pltpu.SemaphoreType.DMA]),
)
def _perm_c(P2, hC, I3, pfbuf, obuf, dbuf, sp, row_v, tot_v, off_v, sem):
    cid = lax.axis_index("c")
    tid = lax.axis_index("s")
    CH2 = NNZ // NS
    HALF = NNZ // NC
    _offsets(hC, row_v, tot_v, off_v, upto=2 * tid)
    lo = cid * HALF
    iota = lax.iota(jnp.int32, 16)

    def sub(s, _):
        base = tid * CH2 + s * SUB
        pltpu.sync_copy(_chunk(P2, base, SUB), pfbuf)

        def body(j, _):
            sl = _vsl(j)
            x = pfbuf[sl]
            d = (x >> 21) & (NBINS - 1)
            dest = _rank(off_v, d) - lo
            inr = (dest >= 0) & (dest < HALF)
            dbuf[sl] = jnp.where(inr, dest, HALF + iota)
            obuf[sl] = x & MASK21
            return 0
        lax.fori_loop(0, VPS, body, 0)
        pltpu.async_copy(obuf, sp.at[dbuf], sem).wait()
        return 0
    lax.fori_loop(0, CH2 // SUB, sub, 0)
    plsc.subcore_barrier()
    SL = HALF // NS
    pltpu.sync_copy(sp.at[pl.ds(tid * SL, SL)],
                    I3.at[pl.ds(pl.multiple_of(lo + tid * SL, 8), SL)])


@functools.partial(
    pl.kernel, mesh=_MESH, compiler_params=_CP,
    out_type=(jax.ShapeDtypeStruct((2 * NNZ,), jnp.int32),
              jax.ShapeDtypeStruct((NNZ,), jnp.int32),
              jax.ShapeDtypeStruct((NNZ,), jnp.float32)),
    scratch_types=[pltpu.VMEM((SUB,), jnp.int32),
                   pltpu.VMEM((SUB,), jnp.int32),
                   pltpu.VMEM((SUB,), jnp.int32),
                   pltpu.VMEM((SUB,), jnp.int32),
                   pltpu.VMEM((SUB,), jnp.float32),
                   pltpu.VMEM((2 * SUB,), jnp.int32),
                   pltpu.SemaphoreType.DMA],
)
def _final(I3, B, R, C, V, EI2, RID, EW, ibuf, bb, rb, cb, vb, eibuf, sem):
    w = _wid()
    iota = lax.iota(jnp.int32, 16)

    def sub(s, _):
        base = w * CHUNK + s * SUB
        pltpu.sync_copy(_chunk(I3, base, SUB), ibuf)
        c0 = pltpu.async_copy(B.at[ibuf], bb, sem)
        c1 = pltpu.async_copy(R.at[ibuf], rb, sem)
        c2 = pltpu.async_copy(C.at[ibuf], cb, sem)
        c3 = pltpu.async_copy(V.at[ibuf], vb, sem)
        c0.wait()
        c1.wait()
        c2.wait()
        c3.wait()

        def ilv(j, _):
            sl = _vsl(j)
            lidx = j * 16 + iota
            plsc.store_scatter(eibuf, [lidx * 2], rb[sl])
            plsc.store_scatter(eibuf, [lidx * 2 + 1], cb[sl])
            return 0
        lax.fori_loop(0, VPS, ilv, 0)
        pltpu.sync_copy(eibuf, _chunk(EI2, base * 2, SUB * 2))
        pltpu.sync_copy(bb, _chunk(RID, base, SUB))
        pltpu.sync_copy(vb, _chunk(EW, base, SUB))
        return 0
    lax.fori_loop(0, NSUB, sub, 0)


def kernel(indices, values):
    B = indices[:, 0]
    R = indices[:, 1]
    C = indices[:, 2]
    hA = _hist_a(B)
    W1, SPL, K2H = _perm_a(B, R, hA)
    hB = _hist_hi(W1)
    P2 = _perm_b(W1, K2H, hB)
    hC = _hist_hi(P2)
    I3 = _perm_c(P2, hC)
    EI2, RID, EW = _final(I3, B, R, C, values)
    return EI2.reshape(NNZ, 2), RID, EW[:, None]


# speedup vs baseline: 2.8288x; 1.0152x over previous
import functools

import jax
import jax.numpy as jnp
from jax import lax
from jax.experimental import pallas as pl
from jax.experimental.pallas import tpu as pltpu
from jax.experimental.pallas import tpu_sc as plsc

NNZ = 2097152
NBINS = 2048
NC = 2
NS = 16
NW = NC * NS
CHUNK = NNZ // NW
SUB = 8192
NSUB = CHUNK // SUB
VPS = SUB // 16
NVB = NBINS // 16
MASK21 = (1 << 21) - 1

_MESH = plsc.VectorSubcoreMesh(core_axis_name="c", subcore_axis_name="s")
_CP = pltpu.CompilerParams(needs_layout_passes=False)


def _wid():
    return lax.axis_index("s") * NC + lax.axis_index("c")


def _vsl(j):
    return pl.ds(pl.multiple_of(j * 16, 16), 16)


def _zero(ref, nv):
    def body(j, _):
        ref[_vsl(j)] = jnp.zeros((16,), jnp.int32)
        return 0
    lax.fori_loop(0, nv, body, 0)


def _chunk(arr, base, n):
    return arr.at[pl.ds(pl.multiple_of(base, 8), n)]


def _hist_common(src_hbm, out_hbm, fbuf, hist_v, dig):
    w = _wid()
    _zero(hist_v, NVB)
    ones = jnp.ones((16,), jnp.int32)

    def sub(s, _):
        pltpu.sync_copy(_chunk(src_hbm, w * CHUNK + s * SUB, SUB), fbuf)

        def body(j, _):
            d = dig(fbuf[_vsl(j)])
            plsc.addupdate_scatter(hist_v, [d], ones)
            return 0
        lax.fori_loop(0, VPS, body, 0)
        return 0
    lax.fori_loop(0, NSUB, sub, 0)
    pltpu.sync_copy(hist_v, out_hbm.at[w])


def _dig_id(x):
    return x & (NBINS - 1)


def _dig_hi(x):
    return (x >> 21) & (NBINS - 1)


def _offsets(hist_hbm, row_v, tot_v, off_v, spl_ref=None, upto=None):
    w = _wid() if upto is None else upto
    _zero(tot_v, NVB)
    _zero(off_v, NVB)

    def trow(t, _):
        pltpu.sync_copy(hist_hbm.at[t], row_v)
        m = (t < w).astype(jnp.int32)

        def inner(j, _):
            sl = _vsl(j)
            row = row_v[sl]
            tot_v[sl] = tot_v[sl] + row
            off_v[sl] = off_v[sl] + row * m
            return 0
        lax.fori_loop(0, NVB, inner, 0)
        return 0
    lax.fori_loop(0, NW, trow, 0)

    fifteen = jnp.full((16,), 15, jnp.int32)

    def scan(j, carry):
        sl = _vsl(j)
        v = tot_v[sl]
        cs = plsc.cumsum(v)
        excl = cs - v + carry
        if spl_ref is not None:
            spl_ref[sl] = excl
        off_v[sl] = off_v[sl] + excl
        return carry + jnp.take(cs, fifteen)
    lax.fori_loop(0, NVB, scan, jnp.zeros((16,), jnp.int32))


def _rank(off_v, d):
    cnt, lastm = plsc.scan_count(d)
    cnt = cnt.astype(jnp.int32)
    cur = plsc.load_gather(off_v, [d])
    plsc.store_scatter(off_v, [d], cur + cnt, mask=lastm)
    return cur + cnt - 1


@functools.partial(
    pl.kernel, mesh=_MESH, compiler_params=_CP,
    out_type=jax.ShapeDtypeStruct((NW, NBINS), jnp.int32),
    scratch_types=[pltpu.VMEM((SUB,), jnp.int32),
                   pltpu.VMEM((NBINS,), jnp.int32)],
)
def _hist_a(B, out, fbuf, hist_v):
    _hist_common(B, out, fbuf, hist_v, _dig_id)


@functools.partial(
    pl.kernel, mesh=_MESH, compiler_params=_CP,
    out_type=jax.ShapeDtypeStruct((NW, NBINS), jnp.int32),
    scratch_types=[pltpu.VMEM((SUB,), jnp.int32),
                   pltpu.VMEM((NBINS,), jnp.int32)],
)
def _hist_hi(X, out, fbuf, hist_v):
    _hist_common(X, out, fbuf, hist_v, _dig_hi)


@functools.partial(
    pl.kernel, mesh=_MESH, compiler_params=_CP,
    out_type=(jax.ShapeDtypeStruct((NNZ,), jnp.int32),
              jax.ShapeDtypeStruct((NBINS,), jnp.int32),
              jax.ShapeDtypeStruct((NNZ,), jnp.int32),
              jax.ShapeDtypeStruct((NW, NBINS), jnp.int32)),
    scratch_types=([pltpu.VMEM((SUB,), jnp.int32)] * 5
                   + [pltpu.VMEM_SHARED((NNZ // NC + 16,), jnp.int32)]
                   + [pltpu.VMEM((NBINS,), jnp.int32)] * 4
                   + [pltpu.SemaphoreType.DMA]),
)
def _perm_a(B, R, hA, W1, SPL, K2H, HB, bbuf, rbuf, wbuf, dbuf, hbuf, sp,
            row_v, tot_v, off_v, spl_v, sem):
    cid = lax.axis_index("c")
    tid = lax.axis_index("s")
    CH2 = NNZ // NS
    HALF = NNZ // NC
    _offsets(hA, row_v, tot_v, off_v, spl_v, upto=2 * tid)
    lo = cid * HALF
    iota = lax.iota(jnp.int32, 16)

    @pl.when(_wid() == 0)
    def _():
        pltpu.sync_copy(spl_v, SPL)

    def sub(s, _):
        base = tid * CH2 + s * SUB
        pltpu.sync_copy(_chunk(B, base, SUB), bbuf)
        pltpu.sync_copy(_chunk(R, base, SUB), rbuf)

        def body(j, _):
            sl = _vsl(j)
            b = bbuf[sl]
            k2 = rbuf[sl] + plsc.load_gather(spl_v, [b])
            hbuf[sl] = k2 >> 11
            dest = _rank(off_v, b) - lo
            inr = (dest >= 0) & (dest < HALF)
            dbuf[sl] = jnp.where(inr, dest, HALF + iota)
            wbuf[sl] = ((k2 & (NBINS - 1)) << 21) | (base + j * 16 + iota)
            return 0
        lax.fori_loop(0, VPS, body, 0)

        @pl.when(cid == 0)
        def _():
            pltpu.sync_copy(hbuf, _chunk(K2H, base, SUB))
        pltpu.async_copy(wbuf, sp.at[dbuf], sem).wait()
        return 0
    lax.fori_loop(0, CH2 // SUB, sub, 0)
    plsc.subcore_barrier()
    SL = HALF // NS
    pltpu.sync_copy(sp.at[pl.ds(tid * SL, SL)],
                    W1.at[pl.ds(pl.multiple_of(lo + tid * SL, 8), SL)])
    _zero(tot_v, NVB)
    ones = jnp.ones((16,), jnp.int32)

    def hsub(s, _):
        pltpu.sync_copy(sp.at[pl.ds(tid * SL + s * SUB, SUB)], bbuf)

        def hbody(j, _):
            plsc.addupdate_scatter(tot_v, [_dig_hi(bbuf[_vsl(j)])], ones)
            return 0
        lax.fori_loop(0, VPS, hbody, 0)
        return 0
    lax.fori_loop(0, SL // SUB, hsub, 0)
    pltpu.sync_copy(tot_v, HB.at[cid * NS + tid])


@functools.partial(
    pl.kernel, mesh=_MESH, compiler_params=_CP,
    out_type=(jax.ShapeDtypeStruct((NNZ,), jnp.int32),
              jax.ShapeDtypeStruct((NW, NBINS), jnp.int32)),
    scratch_types=([pltpu.VMEM((SUB,), jnp.int32)] * 5
                   + [pltpu.VMEM_SHARED((NNZ // NC + 16,), jnp.int32)]
                   + [pltpu.VMEM((NBINS,), jnp.int32)] * 3
                   + [pltpu.SemaphoreType.DMA, pltpu.SemaphoreType.DMA]),
)
def _perm_b(W1, K2H, hB, P2, HC, wbuf, ibuf, hgbuf, pbuf, dbuf, sp,
            row_v, tot_v, off_v, gsem, sem):
    cid = lax.axis_index("c")
    tid = lax.axis_index("s")
    CH2 = NNZ // NS
    HALF = NNZ // NC
    _offsets(hB, row_v, tot_v, off_v, upto=2 * tid)
    lo = cid * HALF
    iota = lax.iota(jnp.int32, 16)

    def sub(s, _):
        base = tid * CH2 + s * SUB
        pltpu.sync_copy(_chunk(W1, base, SUB), wbuf)

        def ext(j, _):
            sl = _vsl(j)
            ibuf[sl] = wbuf[sl] & MASK21
            return 0
        lax.fori_loop(0, VPS, ext, 0)
        pltpu.async_copy(K2H.at[ibuf], hgbuf, gsem).wait()

        def body(j, _):
            sl = _vsl(j)
            d = (wbuf[sl] >> 21) & (NBINS - 1)
            dest = _rank(off_v, d) - lo
            inr = (dest >= 0) & (dest < HALF)
            dbuf[sl] = jnp.where(inr, dest, HALF + iota)
            pbuf[sl] = (hgbuf[sl] << 21) | ibuf[sl]
            return 0
        lax.fori_loop(0, VPS, body, 0)
        pltpu.async_copy(pbuf, sp.at[dbuf], sem).wait()
        return 0
    lax.fori_loop(0, CH2 // SUB, sub, 0)
    plsc.subcore_barrier()
    SL = HALF // NS
    pltpu.sync_copy(sp.at[pl.ds(tid * SL, SL)],
                    P2.at[pl.ds(pl.multiple_of(lo + tid * SL, 8), SL)])
    _zero(tot_v, NVB)
    ones = jnp.ones((16,), jnp.int32)

    def hsub(s, _):
        pltpu.sync_copy(sp.at[pl.ds(tid * SL + s * SUB, SUB)], wbuf)

        def hbody(j, _):
            plsc.addupdate_scatter(tot_v, [_dig_hi(wbuf[_vsl(j)])], ones)
            return 0
        lax.fori_loop(0, VPS, hbody, 0)
        return 0
    lax.fori_loop(0, SL // SUB, hsub, 0)
    pltpu.sync_copy(tot_v, HC.at[cid * NS + tid])


@functools.partial(
    pl.kernel, mesh=_MESH, compiler_params=_CP,
    out_type=(jax.ShapeDtypeStruct((2 * NNZ,), jnp.int32),
              jax.ShapeDtypeStruct((NNZ,), jnp.int32),
              jax.ShapeDtypeStruct((NNZ,), jnp.float32)),
    scratch_types=([pltpu.VMEM((SUB,), jnp.int32)] * 3
                   + [pltpu.VMEM_SHARED((NNZ // NC + 16,), jnp.int32)]
                   + [pltpu.VMEM((NBINS,), jnp.int32)] * 3
                   + [pltpu.VMEM((SUB,), jnp.int32)]
                   + [pltpu.VMEM((SUB,), jnp.float32)]
                   + [pltpu.VMEM((2 * SUB,), jnp.int32)]
                   + [pltpu.SemaphoreType.DMA, pltpu.SemaphoreType.DMA]),
)
def _perm_c(P2, hC, B, R, C, V, EI2, RID, EW, pfbuf, obuf, dbuf, sp,
            row_v, tot_v, off_v, cb, vb, eibuf, sem, gsem):
    bb, rb = obuf, dbuf
    cid = lax.axis_index("c")
    tid = lax.axis_index("s")
    CH2 = NNZ // NS
    HALF = NNZ // NC
    _offsets(hC, row_v, tot_v, off_v, upto=2 * tid)
    lo = cid * HALF
    iota = lax.iota(jnp.int32, 16)

    def sub(s, _):
        base = tid * CH2 + s * SUB
        pltpu.sync_copy(_chunk(P2, base, SUB), pfbuf)

        def body(j, _):
            sl = _vsl(j)
            x = pfbuf[sl]
            d = (x >> 21) & (NBINS - 1)
            dest = _rank(off_v, d) - lo
            inr = (dest >= 0) & (dest < HALF)
            dbuf[sl] = jnp.where(inr, dest, HALF + iota)
            obuf[sl] = x & MASK21
            return 0
        lax.fori_loop(0, VPS, body, 0)
        pltpu.async_copy(obuf, sp.at[dbuf], sem).wait()
        return 0
    lax.fori_loop(0, CH2 // SUB, sub, 0)
    plsc.subcore_barrier()
    SL = HALF // NS

    def fsub(s, _):
        base = lo + tid * SL + s * SUB
        pltpu.sync_copy(sp.at[pl.ds(tid * SL + s * SUB, SUB)], pfbuf)
        c0 = pltpu.async_copy(B.at[pfbuf], bb, gsem)
        c1 = pltpu.async_copy(R.at[pfbuf], rb, gsem)
        c2 = pltpu.async_copy(C.at[pfbuf], cb, gsem)
        c3 = pltpu.async_copy(V.at[pfbuf], vb, gsem)
        c0.wait()
        c1.wait()
        c2.wait()
        c3.wait()

        def ilv(j, _):
            sl = _vsl(j)
            lidx = j * 16 + iota
            plsc.store_scatter(eibuf, [lidx * 2], rb[sl])
            plsc.store_scatter(eibuf, [lidx * 2 + 1], cb[sl])
            return 0
        lax.fori_loop(0, VPS, ilv, 0)
        pltpu.sync_copy(eibuf, _chunk(EI2, base * 2, SUB * 2))
        pltpu.sync_copy(bb, _chunk(RID, base, SUB))
        pltpu.sync_copy(vb, _chunk(EW, base, SUB))
        return 0
    lax.fori_loop(0, SL // SUB, fsub, 0)


def kernel(indices, values):
    B = indices[:, 0]
    R = indices[:, 1]
    C = indices[:, 2]
    hA = _hist_a(B)
    W1, SPL, K2H, hB = _perm_a(B, R, hA)
    P2, hC = _perm_b(W1, K2H, hB)
    EI2, RID, EW = _perm_c(P2, hC, B, R, C, values)
    return EI2.reshape(NNZ, 2), RID, EW[:, None]
